# Initial kernel scaffold; baseline (speedup 1.0000x reference)
import jax
import jax.numpy as jnp
from jax.experimental import pallas as pl


def kernel(x, edge_index, etypes, basis0, coeff0, bias0, basis1, coeff1, bias1, basis2, coeff2, bias2, Wp, bp):
    raise NotImplementedError("write your pallas kernel here")


# speedup vs baseline: 5.5305x; 5.5305x over previous
import functools

import jax
import jax.numpy as jnp
from jax import lax
from jax.experimental import pallas as pl
from jax.experimental.pallas import tpu as pltpu
from jax.experimental.pallas import tpu_sc as plsc

_N = 10000
_E = 320000
_R = 8

_NC = 2
_NS = 16
_L = 16

_NP = 10240
_NRP = _NP * _R
_TRASH = _N

_KI = 128
_CR = 16
_RPT = 160
_EROWS = _RPT * _NS
_EP = _EROWS * _KI

_BN = 1024


def _sc_mesh():
    return plsc.VectorSubcoreMesh(
        core_axis_name="c", subcore_axis_name="s",
        num_cores=_NC, num_subcores=_NS)


def _tc_table(h, warr, bias, act):
    n_parts = warr.shape[0]
    w_part = warr.shape[2] // _R
    nb = _NP // _BN

    def body(h_ref, w_ref, b_ref, o_ref):
        hv = h_ref[...]
        if act:
            hv = jnp.maximum(hv + b_ref[...], 0.0)
        res = jnp.dot(hv, w_ref[0], preferred_element_type=jnp.float32)
        o_ref[...] = res.reshape(1, _BN, _R, w_part)

    d_in = h.shape[1]
    return pl.pallas_call(
        body,
        grid=(n_parts, nb),
        in_specs=[
            pl.BlockSpec((_BN, d_in), lambda c, i: (i, 0)),
            pl.BlockSpec((1, d_in, _R * w_part), lambda c, i: (c, 0, 0)),
            pl.BlockSpec((1, d_in), lambda c, i: (0, 0)),
        ],
        out_specs=pl.BlockSpec((1, _BN, _R, w_part), lambda c, i: (c, i, 0, 0)),
        out_shape=jax.ShapeDtypeStruct((n_parts, _NP, _R, w_part), jnp.float32),
    )(h, warr, bias)


def _tc_score_head(parts, bias, wpp, bpvec):
    d_in = parts.shape[2]
    nb = _NP // _BN

    def body(p_ref, b_ref, w_ref, bp_ref, o_ref):
        hv = p_ref[0] + p_ref[1] + b_ref[...]
        o_ref[...] = (jnp.dot(hv, w_ref[...], preferred_element_type=jnp.float32)
                      + bp_ref[...])

    return pl.pallas_call(
        body,
        grid=(nb,),
        in_specs=[
            pl.BlockSpec((2, _BN, d_in), lambda i: (0, i, 0)),
            pl.BlockSpec((1, d_in), lambda i: (0, 0)),
            pl.BlockSpec((d_in, 2), lambda i: (0, 0)),
            pl.BlockSpec((1, 2), lambda i: (0, 0)),
        ],
        out_specs=pl.BlockSpec((_BN, 2), lambda i: (i, 0)),
        out_shape=jax.ShapeDtypeStruct((_NP, 2), jnp.float32),
    )(parts, bias, wpp, bpvec)


def _sc_edge_agg_fs(table, idx_st, dst_rows):
    w = 128

    @functools.partial(
        pl.kernel,
        out_type=jax.ShapeDtypeStruct((_NP, 2 * w), jnp.float32),
        mesh=_sc_mesh(),
        scratch_types=[
            pltpu.VMEM((_CR, _KI), jnp.int32),
            pltpu.VMEM((_CR, _KI), jnp.int32),
            pltpu.VMEM((_KI, w), jnp.float32),
            pltpu.VMEM_SHARED((_NP, w), jnp.float32),
            pltpu.SemaphoreType.DMA,
        ],
    )
    def agg(table_hbm, idx_hbm, dst_hbm, out_hbm, idx_v, dst_v, rows_v, acc, sem):
        c = lax.axis_index("c")
        s = lax.axis_index("s")
        rbase = s * _RPT

        zv = jnp.zeros((_L,), jnp.float32)

        def zrow(i, _):
            for j in range(w // _L):
                rows_v[i, pl.ds(j * _L, _L)] = zv
            return 0
        lax.fori_loop(0, _KI, zrow, 0)

        zb = s * (_NP // _NS)

        def zcp(k, _):
            pltpu.sync_copy(rows_v, acc.at[pl.ds(zb + k * _KI, _KI)])
            return 0
        lax.fori_loop(0, _NP // _NS // _KI, zcp, 0)
        plsc.subcore_barrier()

        def chunk(t, _):
            rb = rbase + t * _CR
            pltpu.sync_copy(idx_hbm.at[c, pl.ds(rb, _CR)], idx_v)
            pltpu.sync_copy(dst_hbm.at[pl.ds(rb, _CR)], dst_v)

            def step(k, _):
                pltpu.async_copy(table_hbm.at[idx_v.at[k]], rows_v, sem).wait()
                pltpu.sync_copy(rows_v, acc.at[dst_v.at[k]], add=True)
                return 0
            lax.fori_loop(0, _CR, step, 0)
            return 0
        lax.fori_loop(0, _RPT // _CR, chunk, 0)
        plsc.subcore_barrier()

        def ocp(k, _):
            pltpu.sync_copy(
                acc.at[pl.ds(zb + k * _KI, _KI)],
                out_hbm.at[pl.ds(zb + k * _KI, _KI), pl.ds(c * w, w)])
            return 0
        lax.fori_loop(0, _NP // _NS // _KI, ocp, 0)

    return agg(table, idx_st, dst_rows)


def _sc_edge_agg_es(table, idx_st, dst_rows):
    w = 128
    rpt = _EROWS // (_NC * _NS)

    @functools.partial(
        pl.kernel,
        out_type=jax.ShapeDtypeStruct((2, _NP, w), jnp.float32),
        mesh=_sc_mesh(),
        scratch_types=[
            pltpu.VMEM((_CR, _KI), jnp.int32),
            pltpu.VMEM((_CR, _KI), jnp.int32),
            pltpu.VMEM((_KI, w), jnp.float32),
            pltpu.VMEM_SHARED((_NP, w), jnp.float32),
            pltpu.SemaphoreType.DMA,
        ],
    )
    def agg(table_hbm, idx_hbm, dst_hbm, out_hbm, idx_v, dst_v, rows_v, acc, sem):
        c = lax.axis_index("c")
        s = lax.axis_index("s")
        rbase = c * (_EROWS // _NC) + s * rpt

        zv = jnp.zeros((_L,), jnp.float32)

        def zrow(i, _):
            for j in range(w // _L):
                rows_v[i, pl.ds(j * _L, _L)] = zv
            return 0
        lax.fori_loop(0, _KI, zrow, 0)

        zb = s * (_NP // _NS)

        def zcp(k, _):
            pltpu.sync_copy(rows_v, acc.at[pl.ds(zb + k * _KI, _KI)])
            return 0
        lax.fori_loop(0, _NP // _NS // _KI, zcp, 0)
        plsc.subcore_barrier()

        def chunk(t, _):
            rb = rbase + t * _CR
            pltpu.sync_copy(idx_hbm.at[0, pl.ds(rb, _CR)], idx_v)
            pltpu.sync_copy(dst_hbm.at[pl.ds(rb, _CR)], dst_v)

            def step(k, _):
                pltpu.async_copy(table_hbm.at[idx_v.at[k]], rows_v, sem).wait()
                pltpu.sync_copy(rows_v, acc.at[dst_v.at[k]], add=True)
                return 0
            lax.fori_loop(0, _CR, step, 0)
            return 0
        lax.fori_loop(0, rpt // _CR, chunk, 0)
        plsc.subcore_barrier()

        def ocp(k, _):
            pltpu.sync_copy(
                acc.at[pl.ds(zb + k * _KI, _KI)],
                out_hbm.at[c, pl.ds(zb + k * _KI, _KI)])
            return 0
        lax.fori_loop(0, _NP // _NS // _KI, ocp, 0)

    return agg(table, idx_st, dst_rows)


_EPW = _E // (_NC * _NS)


def _sc_score(sa_h, sb_h, src_h, dst_h):
    @functools.partial(
        pl.kernel,
        out_type=jax.ShapeDtypeStruct((_E,), jnp.float32),
        mesh=_sc_mesh(),
        compiler_params=pltpu.CompilerParams(needs_layout_passes=False),
        scratch_types=[
            pltpu.VMEM((_NP,), jnp.float32),
            pltpu.VMEM((_NP,), jnp.float32),
            pltpu.VMEM((_EPW,), jnp.int32),
            pltpu.VMEM((_EPW,), jnp.int32),
            pltpu.VMEM((_EPW,), jnp.float32),
        ],
    )
    def score(sa_hbm, sb_hbm, src_hbm, dst_hbm, out_hbm, sa, sb, srcb, dstb, ob):
        c = lax.axis_index("c")
        s = lax.axis_index("s")
        e0 = (s * _NC + c) * _EPW
        pltpu.sync_copy(sa_hbm, sa)
        pltpu.sync_copy(sb_hbm, sb)
        pltpu.sync_copy(src_hbm.at[pl.ds(e0, _EPW)], srcb)
        pltpu.sync_copy(dst_hbm.at[pl.ds(e0, _EPW)], dstb)

        def step(i, _):
            iv = srcb[pl.ds(i * _L, _L)]
            jv = dstb[pl.ds(i * _L, _L)]
            a = plsc.load_gather(sa, [iv])
            b = plsc.load_gather(sb, [jv])
            ob[pl.ds(i * _L, _L)] = a + b
            return 0
        lax.fori_loop(0, _EPW // _L, step, 0)
        pltpu.sync_copy(ob, out_hbm.at[pl.ds(e0, _EPW)])

    return score(sa_h, sb_h, src_h, dst_h)


def _prep_w(coeff, basis, n_parts):
    wfull = jnp.einsum('rb,bio->rio', coeff, basis)
    w_part = wfull.shape[2] // n_parts
    parts = [
        wfull[:, :, c * w_part:(c + 1) * w_part]
        .transpose(1, 0, 2).reshape(wfull.shape[1], _R * w_part)
        for c in range(n_parts)
    ]
    return jnp.stack(parts)


def kernel(x, edge_index, etypes, basis0, coeff0, bias0, basis1, coeff1,
           bias1, basis2, coeff2, bias2, Wp, bp):
    src = edge_index[0]
    dst = edge_index[1]

    idx = src * _R + etypes
    pad = _EP - _E
    idx_p = jnp.concatenate([idx, jnp.zeros((pad,), jnp.int32)])
    idx_st = jnp.stack([idx_p, idx_p + _NRP]).reshape(2, _EROWS, _KI)
    dst_rows = jnp.concatenate(
        [dst, jnp.full((pad,), _TRASH, jnp.int32)]).reshape(_EROWS, _KI)

    w0 = _prep_w(coeff0, basis0, 2)
    w1 = _prep_w(coeff1, basis1, 2)
    w2 = _prep_w(coeff2, basis2, 1)
    b0 = bias0.reshape(1, -1)
    b1 = bias1.reshape(1, -1)
    b2 = bias2.reshape(1, -1)

    xp = jnp.concatenate(
        [x, jnp.zeros((_NP - _N, x.shape[1]), jnp.float32)])

    t0 = _tc_table(xp, w0, b0, act=False)
    h1 = _sc_edge_agg_fs(t0.reshape(2 * _NRP, 128), idx_st, dst_rows)
    t1 = _tc_table(h1, w1, b0, act=True)
    h2 = _sc_edge_agg_fs(t1.reshape(2 * _NRP, 128), idx_st, dst_rows)
    t2 = _tc_table(h2, w2, b1, act=True)
    h3p = _sc_edge_agg_es(t2.reshape(_NRP, 128), idx_st, dst_rows)

    wpp = jnp.concatenate([Wp[:128], Wp[128:]], axis=1)
    bpvec = jnp.full((1, 2), bp[0] * 0.5, jnp.float32)
    s2 = _tc_score_head(h3p, b2, wpp, bpvec)
    score = _sc_score(s2[:, 0], s2[:, 1], src, dst)
    return score.reshape(_E, 1)
